# baseline (speedup 1.0000x reference)
import functools
import jax
import jax.numpy as jnp
from jax import lax
from jax.experimental import pallas as pl
from jax.experimental.pallas import tpu as pltpu
from jax.experimental.pallas import tpu_sc as plsc

N = 10000
E = 320000
D = 128
H = 16
C = 2

NC = 2
NS = 16
NW = NC * NS
CHUNK = 125
CH = 80
NPAD = 10240
RPT = NPAD // NS

_mesh = plsc.VectorSubcoreMesh(core_axis_name="c", subcore_axis_name="s")
_sc_params = pltpu.CompilerParams(use_tc_tiling_on_sc=False)


@functools.partial(
    pl.kernel,
    out_type=jax.ShapeDtypeStruct((NC, NPAD), jnp.float32),
    mesh=_mesh,
    scratch_types=[
        pltpu.VMEM((CH, CHUNK), jnp.int32),
        pltpu.VMEM((CHUNK,), jnp.float32),
        pltpu.VMEM_SHARED((NPAD,), jnp.float32),
        pltpu.SemaphoreType.DMA,
    ],
    compiler_params=_sc_params,
)
def _sc_degree(ei_hbm, ones_hbm, zeros_hbm, out_hbm, dst_v, ones_v, acc, sem):
    cid = lax.axis_index("c")
    sid = lax.axis_index("s")
    wid = sid * NC + cid
    base = sid * RPT
    pltpu.sync_copy(ei_hbm.at[1, wid], dst_v)
    pltpu.sync_copy(ones_hbm, ones_v)
    pltpu.sync_copy(zeros_hbm.at[pl.ds(base, RPT)], acc.at[pl.ds(base, RPT)])
    plsc.subcore_barrier()

    def body(j, carry):
        pltpu.sync_copy(ones_v, acc.at[dst_v.at[j]], add=True)
        return carry

    lax.fori_loop(0, CH, body, 0)
    plsc.subcore_barrier()
    pltpu.sync_copy(acc.at[pl.ds(base, RPT)], out_hbm.at[cid, pl.ds(base, RPT)])


NBUF = 10
LOOKAHEAD = 5


@functools.partial(
    pl.kernel,
    out_type=jax.ShapeDtypeStruct((NC, NPAD, H), jnp.float32),
    mesh=_mesh,
    scratch_types=[
        pltpu.VMEM((CH, CHUNK), jnp.int32),
        pltpu.VMEM((CH, CHUNK), jnp.int32),
        [pltpu.VMEM((CHUNK, H), jnp.float32) for _ in range(NBUF)],
        pltpu.VMEM_SHARED((NPAD, H), jnp.float32),
        pltpu.VMEM_SHARED((NPAD, H), jnp.float32),
        [pltpu.SemaphoreType.DMA for _ in range(NBUF)],
        [pltpu.SemaphoreType.DMA for _ in range(NBUF)],
    ],
    compiler_params=_sc_params,
)
def _sc_scatter(rows_hbm, ei_hbm, zeros_hbm, out_hbm,
                src_v, dst_v, bufs, acc, rows_sp, gsem, ssem):
    cid = lax.axis_index("c")
    sid = lax.axis_index("s")
    wid = sid * NC + cid
    base = sid * RPT
    pltpu.sync_copy(ei_hbm.at[0, wid], src_v)
    pltpu.sync_copy(ei_hbm.at[1, wid], dst_v)
    pltpu.sync_copy(zeros_hbm.at[pl.ds(base, RPT)], acc.at[pl.ds(base, RPT)])
    pltpu.sync_copy(rows_hbm.at[pl.ds(base, RPT)],
                    rows_sp.at[pl.ds(base, RPT)])
    plsc.subcore_barrier()

    def gather(j, b):
        rows = rows_sp if b % 2 == 0 else rows_hbm
        pltpu.async_copy(rows.at[src_v.at[j]], bufs[b], gsem[b])

    def scatter_desc(j, b):
        return pltpu.make_async_copy(bufs[b], acc.at[dst_v.at[j]], ssem[b])

    for b in range(LOOKAHEAD):
        gather(b, b)

    def body(i, carry):
        for b in range(NBUF):
            j = NBUF * i + b
            rows = rows_sp if b % 2 == 0 else rows_hbm
            pltpu.make_async_copy(rows.at[src_v.at[j]], bufs[b],
                                  gsem[b]).wait()
            pltpu.async_copy(bufs[b], acc.at[dst_v.at[j]], ssem[b], add=True)
            jn = j + LOOKAHEAD
            bn = (b + LOOKAHEAD) % NBUF

            @pl.when(jn < CH)
            def _():
                @pl.when(jn >= NBUF)
                def _():
                    scatter_desc(jn - NBUF, bn).wait()

                gather(jn, bn)
        return carry

    lax.fori_loop(0, CH // NBUF, body, 0)
    for b in range(NBUF):
        scatter_desc(CH - NBUF + b, b).wait()
    plsc.subcore_barrier()
    pltpu.sync_copy(acc.at[pl.ds(base, RPT)],
                    out_hbm.at[cid, pl.ds(base, RPT)])


NP8 = NPAD // 8
NB1 = 9984 // 8


def _tc_mm1_body(xb1_ref, xbt_ref, w1b_ref, h1p_ref):
    h1p_ref[:NB1] = jnp.dot(xb1_ref[...], w1b_ref[...],
                            preferred_element_type=jnp.float32)
    h1p_ref[NB1:N // 8] = jnp.dot(xbt_ref[...], w1b_ref[...],
                                  preferred_element_type=jnp.float32)


def _tc_prep_body(degp_ref, selr_ref, h1p_ref, h1sp_ref, disp_ref):
    degv = degp_ref[0] + degp_ref[1] + 1.0
    disp = jnp.dot(lax.rsqrt(degv), selr_ref[...],
                   preferred_element_type=jnp.float32)
    disp_ref[...] = disp
    h1sp_ref[...] = h1p_ref[...] * disp


def _tc_comb1_body(s1p_ref, h1sp_ref, disp_ref, b1t_ref, gsp_ref):
    disp = disp_ref[...]
    out1 = disp * (s1p_ref[0] + s1p_ref[1] + h1sp_ref[...]) + b1t_ref[...]
    gsp_ref[...] = jnp.maximum(out1, 0.0) * disp


def _tc_final_body(s2p_ref, gsp_ref, disp_ref, w2b_ref, b2t_ref, out_ref):
    o = disp_ref[...] * (s2p_ref[0] + s2p_ref[1] + gsp_ref[...])
    out_ref[...] = jnp.dot(o[:N // 8], w2b_ref[...],
                           preferred_element_type=jnp.float32) + b2t_ref[...]


def _tc_call(body, out_shapes, *args):
    return pl.pallas_call(
        body,
        out_shape=[jax.ShapeDtypeStruct(s, jnp.float32) for s in out_shapes],
    )(*args)


@jax.jit
def kernel(x, edge_index, W1, b1, W2, b2):
    ei = edge_index.astype(jnp.int32).reshape(2, NW, CH, CHUNK)

    ones = jnp.ones((CHUNK,), jnp.float32)
    zeros1 = jnp.zeros((NPAD,), jnp.float32)
    zeros2 = jnp.zeros((NPAD, H), jnp.float32)
    eye8 = jnp.eye(8, dtype=jnp.float32)
    w1b = jnp.kron(eye8, W1)
    w2b = jnp.kron(eye8, W2)
    selr = jnp.kron(eye8, jnp.ones((1, H), jnp.float32))
    b1t = jnp.tile(b1, 8).reshape(1, 8 * H)
    b2t = jnp.tile(b2, 8).reshape(1, 8 * C)
    xb1 = x[:NB1 * 8].reshape(NB1, 8 * D)
    xbt = x[NB1 * 8:].reshape(N // 8 - NB1, 8 * D)

    degp = _sc_degree(ei, ones, zeros1)

    (h1p,) = _tc_call(_tc_mm1_body, [(NP8, 8 * H)], xb1, xbt, w1b)

    h1sp, disp = _tc_call(
        _tc_prep_body, [(NP8, 8 * H), (NP8, 8 * H)],
        degp.reshape(NC, NP8, 8), selr, h1p)

    s1p = _sc_scatter(h1sp.reshape(NPAD, H), ei, zeros2)

    (gsp,) = _tc_call(
        _tc_comb1_body, [(NP8, 8 * H)],
        s1p.reshape(NC, NP8, 8 * H), h1sp, disp, b1t)

    s2p = _sc_scatter(gsp.reshape(NPAD, H), ei, zeros2)

    (out_p,) = _tc_call(
        _tc_final_body, [(N // 8, 8 * C)],
        s2p.reshape(NC, NP8, 8 * H), gsp, disp, w2b, b2t)

    return out_p.reshape(N, C)


# speedup vs baseline: 1.0625x; 1.0625x over previous
import functools
import jax
import jax.numpy as jnp
from jax import lax
from jax.experimental import pallas as pl
from jax.experimental.pallas import tpu as pltpu
from jax.experimental.pallas import tpu_sc as plsc

N = 10000
E = 320000
D = 128
H = 16
C = 2

NC = 2
NS = 16
NW = NC * NS
CHUNK = 125
CH = 80
NPAD = 10240
RPT = NPAD // NS

_mesh = plsc.VectorSubcoreMesh(core_axis_name="c", subcore_axis_name="s")
_sc_params = pltpu.CompilerParams(use_tc_tiling_on_sc=False)


@functools.partial(
    pl.kernel,
    out_type=jax.ShapeDtypeStruct((NC, NPAD), jnp.float32),
    mesh=_mesh,
    scratch_types=[
        pltpu.VMEM((CH, CHUNK), jnp.int32),
        pltpu.VMEM((CHUNK,), jnp.float32),
        pltpu.VMEM_SHARED((NPAD,), jnp.float32),
        pltpu.SemaphoreType.DMA,
    ],
    compiler_params=_sc_params,
)
def _sc_degree(ei_hbm, ones_hbm, zeros_hbm, out_hbm, dst_v, ones_v, acc, sem):
    cid = lax.axis_index("c")
    sid = lax.axis_index("s")
    wid = sid * NC + cid
    base = sid * RPT
    pltpu.sync_copy(ei_hbm.at[1, wid], dst_v)
    pltpu.sync_copy(ones_hbm, ones_v)
    pltpu.sync_copy(zeros_hbm.at[pl.ds(base, RPT)], acc.at[pl.ds(base, RPT)])
    plsc.subcore_barrier()

    def body(j, carry):
        pltpu.sync_copy(ones_v, acc.at[dst_v.at[j]], add=True)
        return carry

    lax.fori_loop(0, CH, body, 0)
    plsc.subcore_barrier()
    pltpu.sync_copy(acc.at[pl.ds(base, RPT)], out_hbm.at[cid, pl.ds(base, RPT)])


NBUF = 10
LOOKAHEAD = 5


@functools.partial(
    pl.kernel,
    out_type=jax.ShapeDtypeStruct((NC, NPAD, H), jnp.float32),
    mesh=_mesh,
    scratch_types=[
        pltpu.VMEM((CH, CHUNK), jnp.int32),
        pltpu.VMEM((CH, CHUNK), jnp.int32),
        [pltpu.VMEM((CHUNK, H), jnp.float32) for _ in range(NBUF)],
        pltpu.VMEM_SHARED((NPAD, H), jnp.float32),
        pltpu.VMEM_SHARED((NPAD, H), jnp.float32),
        [pltpu.SemaphoreType.DMA for _ in range(NBUF)],
        [pltpu.SemaphoreType.DMA for _ in range(NBUF)],
    ],
    compiler_params=_sc_params,
)
def _sc_scatter(rows_hbm, ei_hbm, zeros_hbm, out_hbm,
                src_v, dst_v, bufs, acc, rows_sp, gsem, ssem):
    cid = lax.axis_index("c")
    sid = lax.axis_index("s")
    wid = sid * NC + cid
    base = sid * RPT
    pltpu.sync_copy(ei_hbm.at[0, wid], src_v)
    pltpu.sync_copy(ei_hbm.at[1, wid], dst_v)
    pltpu.sync_copy(zeros_hbm.at[pl.ds(base, RPT)], acc.at[pl.ds(base, RPT)])
    pltpu.sync_copy(rows_hbm.at[pl.ds(base, RPT)],
                    rows_sp.at[pl.ds(base, RPT)])
    plsc.subcore_barrier()

    def gather(j, b):
        pltpu.async_copy(rows_sp.at[src_v.at[j]], bufs[b], gsem[b])

    def scatter_desc(j, b):
        return pltpu.make_async_copy(bufs[b], acc.at[dst_v.at[j]], ssem[b])

    for b in range(LOOKAHEAD):
        gather(b, b)

    def body(i, carry):
        for b in range(NBUF):
            j = NBUF * i + b
            pltpu.make_async_copy(rows_sp.at[src_v.at[j]], bufs[b],
                                  gsem[b]).wait()
            pltpu.async_copy(bufs[b], acc.at[dst_v.at[j]], ssem[b], add=True)
            jn = j + LOOKAHEAD
            bn = (b + LOOKAHEAD) % NBUF

            @pl.when(jn < CH)
            def _():
                @pl.when(jn >= NBUF)
                def _():
                    scatter_desc(jn - NBUF, bn).wait()

                gather(jn, bn)
        return carry

    lax.fori_loop(0, CH // NBUF, body, 0)
    for b in range(NBUF):
        scatter_desc(CH - NBUF + b, b).wait()
    plsc.subcore_barrier()
    pltpu.sync_copy(acc.at[pl.ds(base, RPT)],
                    out_hbm.at[cid, pl.ds(base, RPT)])


NP8 = NPAD // 8
NB1 = 9984 // 8


def _tc_mm1_body(xb1_ref, xbt_ref, w1b_ref, h1p_ref):
    h1p_ref[:NB1] = jnp.dot(xb1_ref[...], w1b_ref[...],
                            preferred_element_type=jnp.float32)
    h1p_ref[NB1:N // 8] = jnp.dot(xbt_ref[...], w1b_ref[...],
                                  preferred_element_type=jnp.float32)


def _tc_prep_body(degp_ref, selr_ref, h1p_ref, h1sp_ref, disp_ref):
    degv = degp_ref[0] + degp_ref[1] + 1.0
    disp = jnp.dot(lax.rsqrt(degv), selr_ref[...],
                   preferred_element_type=jnp.float32)
    disp_ref[...] = disp
    h1sp_ref[...] = h1p_ref[...] * disp


def _tc_comb1_body(s1p_ref, h1sp_ref, disp_ref, b1t_ref, gsp_ref):
    disp = disp_ref[...]
    out1 = disp * (s1p_ref[0] + s1p_ref[1] + h1sp_ref[...]) + b1t_ref[...]
    gsp_ref[...] = jnp.maximum(out1, 0.0) * disp


def _tc_final_body(s2p_ref, gsp_ref, disp_ref, w2b_ref, b2t_ref, out_ref):
    o = disp_ref[...] * (s2p_ref[0] + s2p_ref[1] + gsp_ref[...])
    out_ref[...] = jnp.dot(o[:N // 8], w2b_ref[...],
                           preferred_element_type=jnp.float32) + b2t_ref[...]


def _tc_call(body, out_shapes, *args):
    return pl.pallas_call(
        body,
        out_shape=[jax.ShapeDtypeStruct(s, jnp.float32) for s in out_shapes],
    )(*args)


@jax.jit
def kernel(x, edge_index, W1, b1, W2, b2):
    ei = edge_index.astype(jnp.int32).reshape(2, NW, CH, CHUNK)

    ones = jnp.ones((CHUNK,), jnp.float32)
    zeros1 = jnp.zeros((NPAD,), jnp.float32)
    zeros2 = jnp.zeros((NPAD, H), jnp.float32)
    eye8 = jnp.eye(8, dtype=jnp.float32)
    w1b = jnp.kron(eye8, W1)
    w2b = jnp.kron(eye8, W2)
    selr = jnp.kron(eye8, jnp.ones((1, H), jnp.float32))
    b1t = jnp.tile(b1, 8).reshape(1, 8 * H)
    b2t = jnp.tile(b2, 8).reshape(1, 8 * C)
    xb1 = x[:NB1 * 8].reshape(NB1, 8 * D)
    xbt = x[NB1 * 8:].reshape(N // 8 - NB1, 8 * D)

    degp = _sc_degree(ei, ones, zeros1)

    (h1p,) = _tc_call(_tc_mm1_body, [(NP8, 8 * H)], xb1, xbt, w1b)

    h1sp, disp = _tc_call(
        _tc_prep_body, [(NP8, 8 * H), (NP8, 8 * H)],
        degp.reshape(NC, NP8, 8), selr, h1p)

    s1p = _sc_scatter(h1sp.reshape(NPAD, H), ei, zeros2)

    (gsp,) = _tc_call(
        _tc_comb1_body, [(NP8, 8 * H)],
        s1p.reshape(NC, NP8, 8 * H), h1sp, disp, b1t)

    s2p = _sc_scatter(gsp.reshape(NPAD, H), ei, zeros2)

    (out_p,) = _tc_call(
        _tc_final_body, [(N // 8, 8 * C)],
        s2p.reshape(NC, NP8, 8 * H), gsp, disp, w2b, b2t)

    return out_p.reshape(N, C)
